# Initial kernel scaffold; baseline (speedup 1.0000x reference)
import jax
import jax.numpy as jnp
from jax.experimental import pallas as pl


def kernel(data, length, embed_table, W, b):
    raise NotImplementedError("write your pallas kernel here")


# speedup vs baseline: 1.7822x; 1.7822x over previous
import functools

import jax
import jax.numpy as jnp
from jax import lax
from jax.experimental import pallas as pl
from jax.experimental.pallas import tpu as pltpu
from jax.experimental.pallas import tpu_sc as plsc

VOCAB = 1000000
EMB = 64
B = 16384
L = 200
NCLS = 20

NC = 2
NS = 16
LANES = 16
NW = NC * NS
ROWS_PER_W = B // NW
C0, C1 = 104, 96
NVEC = EMB // LANES


def _sc_pool(data_flat, table):
    mesh = plsc.VectorSubcoreMesh(
        core_axis_name="c", subcore_axis_name="s", num_cores=NC, num_subcores=NS
    )

    @functools.partial(
        pl.kernel,
        out_type=jax.ShapeDtypeStruct((B * EMB,), jnp.float32),
        mesh=mesh,
        compiler_params=pltpu.CompilerParams(use_tc_tiling_on_sc=False),
        scratch_types=[
            pltpu.VMEM((L,), jnp.int32),
            pltpu.VMEM((L, EMB), jnp.float32),
            pltpu.VMEM((EMB,), jnp.float32),
            pltpu.SemaphoreType.DMA,
            pltpu.SemaphoreType.DMA,
        ],
    )
    def k(data_hbm, table_hbm, out_hbm, idx_v, rows_v, ostage, sem1, sem2):
        wid = lax.axis_index("s") * NC + lax.axis_index("c")
        base = wid * ROWS_PER_W

        def row_body(g, carry):
            row = base + g
            pltpu.sync_copy(data_hbm.at[pl.ds(row * L, L)], idx_v)
            cp1 = pltpu.async_copy(
                table_hbm.at[idx_v.at[pl.ds(0, C0)]], rows_v.at[pl.ds(0, C0)], sem1
            )
            cp2 = pltpu.async_copy(
                table_hbm.at[idx_v.at[pl.ds(C0, C1)]], rows_v.at[pl.ds(C0, C1)], sem2
            )
            cp1.wait()
            cp2.wait()

            def acc_body(j, accs):
                return tuple(
                    accs[t] + rows_v[j, pl.ds(t * LANES, LANES)] for t in range(NVEC)
                )

            accs = tuple(jnp.zeros((LANES,), jnp.float32) for _ in range(NVEC))
            accs = lax.fori_loop(0, L, acc_body, accs)
            for t in range(NVEC):
                ostage[pl.ds(t * LANES, LANES)] = accs[t]
            pltpu.sync_copy(ostage, out_hbm.at[pl.ds(row * EMB, EMB)])
            return carry

        lax.fori_loop(0, ROWS_PER_W, row_body, 0)

    return k(data_flat, table)


def _tc_linear(sums, inv_len, W2, b2):
    BLK = 2048

    def body(s_ref, l_ref, w_ref, b_ref, o_ref):
        pooled = s_ref[...] * l_ref[...]
        o_ref[...] = (
            lax.dot_general(
                pooled, w_ref[...], (((1,), (1,)), ((), ())),
                preferred_element_type=jnp.float32,
            )
            + b_ref[...]
        )

    return pl.pallas_call(
        body,
        grid=(B // BLK,),
        in_specs=[
            pl.BlockSpec((BLK, EMB), lambda i: (i, 0)),
            pl.BlockSpec((BLK, 1), lambda i: (i, 0)),
            pl.BlockSpec((NCLS, EMB), lambda i: (0, 0)),
            pl.BlockSpec((1, NCLS), lambda i: (0, 0)),
        ],
        out_specs=pl.BlockSpec((BLK, NCLS), lambda i: (i, 0)),
        out_shape=jax.ShapeDtypeStruct((B, NCLS), jnp.float32),
    )(sums, inv_len, W2, b2)


def kernel(data, length, embed_table, W, b):
    data_flat = data.reshape(B * L).astype(jnp.int32)
    sums = _sc_pool(data_flat, embed_table).reshape(B, EMB)
    inv_len = (1.0 / length.astype(jnp.float32)).reshape(B, 1)
    return _tc_linear(sums, inv_len, W, b.reshape(1, NCLS))


# speedup vs baseline: 3.4163x; 1.9169x over previous
import functools

import jax
import jax.numpy as jnp
from jax import lax
from jax.experimental import pallas as pl
from jax.experimental.pallas import tpu as pltpu
from jax.experimental.pallas import tpu_sc as plsc

VOCAB = 1000000
EMB = 64
B = 16384
L = 200
NCLS = 20

NC = 2
NS = 16
LANES = 16
NW = NC * NS
ROWS_PER_W = B // NW
C0, C1 = 104, 96
NVEC = EMB // LANES
RPP = 128
NPH = ROWS_PER_W // RPP
NRING = 4
UNROLL = 8


def _sc_pool(data_flat, table):
    mesh = plsc.VectorSubcoreMesh(
        core_axis_name="c", subcore_axis_name="s", num_cores=NC, num_subcores=NS
    )

    @functools.partial(
        pl.kernel,
        out_type=jax.ShapeDtypeStruct((B, EMB), jnp.float32),
        mesh=mesh,
        compiler_params=pltpu.CompilerParams(use_tc_tiling_on_sc=False),
        scratch_types=[
            pltpu.VMEM((2, RPP * L), jnp.int32),
            pltpu.VMEM((NRING, L, EMB), jnp.float32),
            pltpu.VMEM((2, RPP, EMB), jnp.float32),
            pltpu.SemaphoreType.DMA,
            pltpu.SemaphoreType.DMA,
            pltpu.SemaphoreType.DMA,
            pltpu.SemaphoreType.DMA,
            pltpu.SemaphoreType.DMA,
            pltpu.SemaphoreType.DMA,
            pltpu.SemaphoreType.DMA,
            pltpu.SemaphoreType.DMA,
        ],
    )
    def k(data_hbm, table_hbm, out_hbm, idxg, rows, ostage,
          is0, is1, g0, g1, g2, g3, o0, o1):
        isem = (is0, is1)
        gsem = (g0, g1, g2, g3)
        osem = (o0, o1)
        wid = lax.axis_index("s") * NC + lax.axis_index("c")
        base = wid * ROWS_PER_W

        def issue_idx(p, pp):
            return pltpu.async_copy(
                data_hbm.at[pl.ds((base + p * RPP) * L, RPP * L)],
                idxg.at[pp], isem[pp])

        def issue_gathers(idx_p, roff, slot):
            off = roff * L
            pltpu.async_copy(
                table_hbm.at[idx_p.at[pl.ds(off, C0)]],
                rows.at[slot].at[pl.ds(0, C0)], gsem[slot])
            pltpu.async_copy(
                table_hbm.at[idx_p.at[pl.ds(off + C0, C1)]],
                rows.at[slot].at[pl.ds(C0, C1)], gsem[slot])

        def wait_gathers(slot):
            pltpu.make_async_copy(
                table_hbm.at[pl.ds(0, L)], rows.at[slot], gsem[slot]).wait()

        def accumulate(slot):
            slot_ref = rows.at[slot]

            def body(jj, accs):
                accs = list(accs)
                for u in range(UNROLL):
                    j = jj * UNROLL + u
                    for t in range(NVEC):
                        accs[t] = accs[t] + slot_ref[j, pl.ds(t * LANES, LANES)]
                return tuple(accs)

            accs = tuple(jnp.zeros((LANES,), jnp.float32) for _ in range(NVEC))
            return lax.fori_loop(0, L // UNROLL, body, accs)

        def store_row(opp, r, accs):
            for t in range(NVEC):
                opp[r, pl.ds(t * LANES, LANES)] = accs[t]

        idesc = [issue_idx(0, 0), None]
        odesc = [None, None]
        for p in range(NPH):
            pp = p % 2
            if odesc[pp] is not None:
                odesc[pp].wait()
            idesc[pp].wait()
            if p + 1 < NPH:
                idesc[(p + 1) % 2] = issue_idx(p + 1, (p + 1) % 2)
            idx_p = idxg.at[pp]
            opp = ostage.at[pp]
            for s in range(NRING):
                issue_gathers(idx_p, s, s)

            def inner(h, carry, idx_p=idx_p, opp=opp):
                for j in range(NRING):
                    r = NRING * h + j
                    wait_gathers(j)
                    accs = accumulate(j)
                    store_row(opp, r, accs)
                    issue_gathers(idx_p, r + NRING, j)
                return carry

            lax.fori_loop(0, RPP // NRING - 1, inner, 0)
            for j in range(NRING):
                r = RPP - NRING + j
                wait_gathers(j)
                accs = accumulate(j)
                store_row(opp, r, accs)
            odesc[pp] = pltpu.async_copy(
                opp, out_hbm.at[pl.ds(base + p * RPP, RPP)], osem[pp])
        odesc[0].wait()
        odesc[1].wait()

    return k(data_flat, table)


def _tc_linear(sums, inv_len, W2, b2):
    BLK = 2048

    def body(s_ref, l_ref, w_ref, b_ref, o_ref):
        pooled = s_ref[...] * l_ref[...]
        o_ref[...] = (
            lax.dot_general(
                pooled, w_ref[...], (((1,), (1,)), ((), ())),
                preferred_element_type=jnp.float32,
            )
            + b_ref[...]
        )

    return pl.pallas_call(
        body,
        grid=(B // BLK,),
        in_specs=[
            pl.BlockSpec((BLK, EMB), lambda i: (i, 0)),
            pl.BlockSpec((BLK, 1), lambda i: (i, 0)),
            pl.BlockSpec((NCLS, EMB), lambda i: (0, 0)),
            pl.BlockSpec((1, NCLS), lambda i: (0, 0)),
        ],
        out_specs=pl.BlockSpec((BLK, NCLS), lambda i: (i, 0)),
        out_shape=jax.ShapeDtypeStruct((B, NCLS), jnp.float32),
    )(sums, inv_len, W2, b2)


def kernel(data, length, embed_table, W, b):
    data_flat = data.reshape(B * L).astype(jnp.int32)
    sums = _sc_pool(data_flat, embed_table)
    inv_len = (1.0 / length.astype(jnp.float32)).reshape(B, 1)
    return _tc_linear(sums, inv_len, W, b.reshape(1, NCLS))
